# baseline (speedup 1.0000x reference)
import jax
import jax.numpy as jnp
from jax import lax
from jax.experimental import pallas as pl
from jax.experimental.pallas import tpu as pltpu

VOCAB = 100000
EMBED = 128
CONTEXT = 200
HIDDEN = 128

K_BLK = 12800
NK = (CONTEXT * EMBED) // K_BLK
V_BLK = 33408
NV = -(-VOCAB // V_BLK)
LAST_ROWS = VOCAB - (NV - 1) * V_BLK
P1 = NK
P2 = NK + NV
NSTEPS = NK + 2 * NV
RING = 2
ROWS_PER_STEP = CONTEXT // NK


def _fused(idx, table, W1, b1, W2T, b2):
    def body(idx_ref, table_ref, w2t_ref, w1_ref, b1_ref, b2_ref, out_ref,
             e_ref, acc_ref, logits_ref, w2buf, b2v_ref, m_ref, s_ref,
             nissued_ref, gsem, w2sem, b2sem):
        i = pl.program_id(0)

        def w2_dma(nb, slot):
            return pltpu.make_async_copy(
                w2t_ref.at[pl.ds(pl.multiple_of(nb * V_BLK, 8), V_BLK), :],
                w2buf.at[slot],
                w2sem.at[slot])

        def w2_dma_last(slot):
            return pltpu.make_async_copy(
                w2t_ref.at[pl.ds((NV - 1) * V_BLK, LAST_ROWS), :],
                w2buf.at[slot, pl.ds(0, LAST_ROWS), :],
                w2sem.at[slot])

        def maybe_issue(consumed):
            for _ in range(2):
                nb = nissued_ref[0]
                ok = (nb < NV) & (nb < consumed + RING)

                @pl.when(ok & (nb < NV - 1))
                def _():
                    w2_dma(nb, lax.rem(nb, RING)).start()

                @pl.when(ok & (nb == NV - 1))
                def _():
                    w2_dma_last(lax.rem(nb, RING)).start()

                @pl.when(ok)
                def _():
                    nissued_ref[0] = nb + 1

        def row_dma(j):
            row = idx_ref[j]
            return pltpu.make_async_copy(
                table_ref.at[pl.ds(row, 1), :],
                e_ref.at[0:1, pl.ds(pl.multiple_of(j * EMBED, 128), EMBED)],
                gsem)

        @pl.when(i == 0)
        def _():
            nissued_ref[0] = 0
            lax.fori_loop(0, ROWS_PER_STEP, lambda j, _: (row_dma(j).start(), 0)[1], 0)
            pltpu.make_async_copy(b2_ref, b2v_ref, b2sem).start()
            maybe_issue(0)
            lax.fori_loop(ROWS_PER_STEP, CONTEXT,
                          lambda j, _: (row_dma(j).start(), 0)[1], 0)

        @pl.when(i < P1)
        def _():
            @pl.when(i == 0)
            def _():
                acc_ref[...] = jnp.zeros_like(acc_ref)

            @pl.when(i > 0)
            def _():
                maybe_issue(0)

            lax.fori_loop(i * ROWS_PER_STEP, (i + 1) * ROWS_PER_STEP,
                          lambda j, _: (row_dma(j).wait(), 0)[1], 0)

            e_blk = e_ref[0:1, pl.ds(pl.multiple_of(i * K_BLK, 128), K_BLK)]
            acc_ref[...] += jnp.dot(e_blk, w1_ref[...],
                                    preferred_element_type=jnp.float32)

            @pl.when(i == P1 - 1)
            def _():
                acc_ref[...] = jnp.maximum(acc_ref[...] + b1_ref[...], 0.0)

        @pl.when((i >= P1) & (i < P2))
        def _():
            j = i - P1
            slot = lax.rem(j, RING)

            @pl.when(j == 0)
            def _():
                pltpu.make_async_copy(b2_ref, b2v_ref, b2sem).wait()

            @pl.when(j < NV - 1)
            def _():
                w2_dma(j, slot).wait()

            @pl.when(j == NV - 1)
            def _():
                w2_dma_last(slot).wait()

            w2_blk = w2buf[slot]
            b2_blk = b2v_ref[pl.ds(pl.multiple_of(j * V_BLK, 128), V_BLK)]
            z = lax.dot_general(acc_ref[...], w2_blk,
                                (((1,), (1,)), ((), ())),
                                preferred_element_type=jnp.float32) \
                + b2_blk
            logits_ref[pl.ds(j, 1), :] = z
            col = j * V_BLK + lax.broadcasted_iota(jnp.int32, (1, V_BLK), 1)
            zm = jnp.where(col < VOCAB, z, -jnp.inf)
            bm = jnp.max(zm)

            @pl.when(j == 0)
            def _():
                m_ref[0] = bm
                s_ref[0] = jnp.sum(jnp.exp(zm - bm))

            @pl.when(j > 0)
            def _():
                m_old = m_ref[0]
                new_m = jnp.maximum(m_old, bm)
                s_ref[0] = s_ref[0] * jnp.exp(m_old - new_m) + \
                    jnp.sum(jnp.exp(zm - new_m))
                m_ref[0] = new_m

            maybe_issue(j + 1)

        @pl.when(i >= P2)
        def _():
            j = i - P2
            norm = m_ref[0] + jnp.log(s_ref[0])
            out_ref[...] = logits_ref[pl.ds(j, 1), :] - norm

    return pl.pallas_call(
        body,
        grid=(NSTEPS,),
        in_specs=[
            pl.BlockSpec(memory_space=pltpu.SMEM),
            pl.BlockSpec(memory_space=pl.ANY),
            pl.BlockSpec(memory_space=pl.ANY),
            pl.BlockSpec((K_BLK, HIDDEN),
                         lambda i: (jnp.minimum(i, P1 - 1), 0)),
            pl.BlockSpec((HIDDEN,), lambda i: (0,)),
            pl.BlockSpec(memory_space=pl.ANY),
        ],
        out_specs=pl.BlockSpec((1, V_BLK),
                               lambda i: (0, jnp.clip(i - P2, 0, NV - 1))),
        out_shape=jax.ShapeDtypeStruct((1, VOCAB), jnp.float32),
        scratch_shapes=[
            pltpu.VMEM((1, CONTEXT * EMBED), jnp.float32),
            pltpu.VMEM((1, HIDDEN), jnp.float32),
            pltpu.VMEM((NV, V_BLK), jnp.float32),
            pltpu.VMEM((RING, V_BLK, HIDDEN), jnp.float32),
            pltpu.VMEM((VOCAB,), jnp.float32),
            pltpu.SMEM((1,), jnp.float32),
            pltpu.SMEM((1,), jnp.float32),
            pltpu.SMEM((1,), jnp.int32),
            pltpu.SemaphoreType.DMA,
            pltpu.SemaphoreType.DMA((RING,)),
            pltpu.SemaphoreType.DMA,
        ],
    )(idx, table, W2T, W1, b1, b2)


def kernel(idx, table, W1, b1, W2, b2):
    return _fused(idx.astype(jnp.int32), table, W1, b1, W2.T, b2)


# speedup vs baseline: 1.1812x; 1.1812x over previous
import jax
import jax.numpy as jnp
from jax import lax
from jax.experimental import pallas as pl
from jax.experimental.pallas import tpu as pltpu

VOCAB = 100000
EMBED = 128
CONTEXT = 200
HIDDEN = 128

K_BLK = 12800
NK = (CONTEXT * EMBED) // K_BLK
V_BLK = 25088
NV = -(-VOCAB // V_BLK)
LAST_ROWS = VOCAB - (NV - 1) * V_BLK
P1 = NK
P2 = NK + NV
NSTEPS = NK + 2 * NV
RING = 3
ROWS_PER_STEP = CONTEXT // NK


def _fused(idx, table, W1, b1, W2T, b2):
    def body(idx_ref, table_ref, w2t_ref, w1_ref, b1_ref, b2_ref, out_ref,
             e_ref, acc_ref, logits_ref, w2buf, b2v_ref, m_ref, s_ref,
             nissued_ref, gsem, w2sem, b2sem):
        i = pl.program_id(0)

        def w2_dma(nb, slot):
            return pltpu.make_async_copy(
                w2t_ref.at[pl.ds(pl.multiple_of(nb * V_BLK, 8), V_BLK), :],
                w2buf.at[slot],
                w2sem.at[slot])

        def w2_dma_last(slot):
            return pltpu.make_async_copy(
                w2t_ref.at[pl.ds((NV - 1) * V_BLK, LAST_ROWS), :],
                w2buf.at[slot, pl.ds(0, LAST_ROWS), :],
                w2sem.at[slot])

        def maybe_issue(consumed):
            for _ in range(2):
                nb = nissued_ref[0]
                ok = (nb < NV) & (nb < consumed + RING)

                @pl.when(ok & (nb < NV - 1))
                def _():
                    w2_dma(nb, lax.rem(nb, RING)).start()

                @pl.when(ok & (nb == NV - 1))
                def _():
                    w2_dma_last(lax.rem(nb, RING)).start()

                @pl.when(ok)
                def _():
                    nissued_ref[0] = nb + 1

        def row_dma(j):
            row = idx_ref[j]
            return pltpu.make_async_copy(
                table_ref.at[pl.ds(row, 1), :],
                e_ref.at[0:1, pl.ds(pl.multiple_of(j * EMBED, 128), EMBED)],
                gsem)

        @pl.when(i == 0)
        def _():
            nissued_ref[0] = 0
            lax.fori_loop(0, ROWS_PER_STEP, lambda j, _: (row_dma(j).start(), 0)[1], 0)
            pltpu.make_async_copy(b2_ref, b2v_ref, b2sem).start()
            maybe_issue(0)
            lax.fori_loop(ROWS_PER_STEP, CONTEXT,
                          lambda j, _: (row_dma(j).start(), 0)[1], 0)

        @pl.when(i < P1)
        def _():
            @pl.when(i == 0)
            def _():
                acc_ref[...] = jnp.zeros_like(acc_ref)

            @pl.when(i > 0)
            def _():
                maybe_issue(0)

            lax.fori_loop(i * ROWS_PER_STEP, (i + 1) * ROWS_PER_STEP,
                          lambda j, _: (row_dma(j).wait(), 0)[1], 0)

            e_blk = e_ref[0:1, pl.ds(pl.multiple_of(i * K_BLK, 128), K_BLK)]
            acc_ref[...] += jnp.dot(e_blk, w1_ref[...],
                                    preferred_element_type=jnp.float32)

            @pl.when(i == P1 - 1)
            def _():
                acc_ref[...] = jnp.maximum(acc_ref[...] + b1_ref[...], 0.0)

        @pl.when((i >= P1) & (i < P2))
        def _():
            j = i - P1
            slot = lax.rem(j, RING)

            @pl.when(j == 0)
            def _():
                pltpu.make_async_copy(b2_ref, b2v_ref, b2sem).wait()

            @pl.when(j < NV - 1)
            def _():
                w2_dma(j, slot).wait()

            @pl.when(j == NV - 1)
            def _():
                w2_dma_last(slot).wait()

            w2_blk = w2buf[slot]
            b2_blk = b2v_ref[pl.ds(pl.multiple_of(j * V_BLK, 128), V_BLK)]
            z = lax.dot_general(acc_ref[...], w2_blk,
                                (((1,), (1,)), ((), ())),
                                preferred_element_type=jnp.float32) \
                + b2_blk
            logits_ref[pl.ds(j, 1), :] = z
            col = j * V_BLK + lax.broadcasted_iota(jnp.int32, (1, V_BLK), 1)
            zm = jnp.where(col < VOCAB, z, -jnp.inf)
            bm = jnp.max(zm)

            @pl.when(j == 0)
            def _():
                m_ref[0] = bm
                s_ref[0] = jnp.sum(jnp.exp(zm - bm))

            @pl.when(j > 0)
            def _():
                m_old = m_ref[0]
                new_m = jnp.maximum(m_old, bm)
                s_ref[0] = s_ref[0] * jnp.exp(m_old - new_m) + \
                    jnp.sum(jnp.exp(zm - new_m))
                m_ref[0] = new_m

            maybe_issue(j + 1)

        @pl.when(i >= P2)
        def _():
            j = i - P2
            norm = m_ref[0] + jnp.log(s_ref[0])
            out_ref[...] = logits_ref[pl.ds(j, 1), :] - norm

    return pl.pallas_call(
        body,
        grid=(NSTEPS,),
        in_specs=[
            pl.BlockSpec(memory_space=pltpu.SMEM),
            pl.BlockSpec(memory_space=pl.ANY),
            pl.BlockSpec(memory_space=pl.ANY),
            pl.BlockSpec((K_BLK, HIDDEN),
                         lambda i: (jnp.minimum(i, P1 - 1), 0)),
            pl.BlockSpec((HIDDEN,), lambda i: (0,)),
            pl.BlockSpec(memory_space=pl.ANY),
        ],
        out_specs=pl.BlockSpec((1, V_BLK),
                               lambda i: (0, jnp.clip(i - P2, 0, NV - 1))),
        out_shape=jax.ShapeDtypeStruct((1, VOCAB), jnp.float32),
        scratch_shapes=[
            pltpu.VMEM((1, CONTEXT * EMBED), jnp.float32),
            pltpu.VMEM((1, HIDDEN), jnp.float32),
            pltpu.VMEM((NV, V_BLK), jnp.float32),
            pltpu.VMEM((RING, V_BLK, HIDDEN), jnp.float32),
            pltpu.VMEM((VOCAB,), jnp.float32),
            pltpu.SMEM((1,), jnp.float32),
            pltpu.SMEM((1,), jnp.float32),
            pltpu.SMEM((1,), jnp.int32),
            pltpu.SemaphoreType.DMA,
            pltpu.SemaphoreType.DMA((RING,)),
            pltpu.SemaphoreType.DMA,
        ],
    )(idx, table, W2T, W1, b1, b2)


def kernel(idx, table, W1, b1, W2, b2):
    return _fused(idx.astype(jnp.int32), table, W1, b1, W2.T, b2)


# speedup vs baseline: 1.2339x; 1.0446x over previous
import jax
import jax.numpy as jnp
from jax import lax
from jax.experimental import pallas as pl
from jax.experimental.pallas import tpu as pltpu

VOCAB = 100000
EMBED = 128
CONTEXT = 200
HIDDEN = 128

K_BLK = 12800
NK = (CONTEXT * EMBED) // K_BLK
V_BLK = 25088
NV = -(-VOCAB // V_BLK)
LAST_ROWS = VOCAB - (NV - 1) * V_BLK
P1 = NK
NSTEPS = NK + NV
RING = 3
ROWS_PER_STEP = CONTEXT // NK


def _fused(idx, table, W1, b1, W2T, b2):
    def body(idx_ref, table_ref, w2t_ref, w1_ref, b1_ref, b2_ref, out_ref,
             e_ref, acc_ref, logits_ref, w2buf, b2v_ref, m_ref, s_ref,
             nissued_ref, gsem, w2sem, b2sem):
        i = pl.program_id(0)

        def w2_dma(nb, slot):
            return pltpu.make_async_copy(
                w2t_ref.at[pl.ds(pl.multiple_of(nb * V_BLK, 8), V_BLK), :],
                w2buf.at[slot],
                w2sem.at[slot])

        def w2_dma_last(slot):
            return pltpu.make_async_copy(
                w2t_ref.at[pl.ds((NV - 1) * V_BLK, LAST_ROWS), :],
                w2buf.at[slot, pl.ds(0, LAST_ROWS), :],
                w2sem.at[slot])

        def maybe_issue(consumed):
            for _ in range(2):
                nb = nissued_ref[0]
                ok = (nb < NV) & (nb < consumed + RING)

                @pl.when(ok & (nb < NV - 1))
                def _():
                    w2_dma(nb, lax.rem(nb, RING)).start()

                @pl.when(ok & (nb == NV - 1))
                def _():
                    w2_dma_last(lax.rem(nb, RING)).start()

                @pl.when(ok)
                def _():
                    nissued_ref[0] = nb + 1

        def row_dma(j):
            row = idx_ref[j]
            return pltpu.make_async_copy(
                table_ref.at[pl.ds(row, 1), :],
                e_ref.at[0:1, pl.ds(pl.multiple_of(j * EMBED, 128), EMBED)],
                gsem)

        @pl.when(i == 0)
        def _():
            nissued_ref[0] = 0
            lax.fori_loop(0, ROWS_PER_STEP, lambda j, _: (row_dma(j).start(), 0)[1], 0)
            pltpu.make_async_copy(b2_ref, b2v_ref, b2sem).start()
            maybe_issue(0)
            lax.fori_loop(ROWS_PER_STEP, CONTEXT,
                          lambda j, _: (row_dma(j).start(), 0)[1], 0)

        @pl.when(i < P1)
        def _():
            @pl.when(i == 0)
            def _():
                acc_ref[...] = jnp.zeros_like(acc_ref)

            @pl.when(i > 0)
            def _():
                maybe_issue(0)

            lax.fori_loop(i * ROWS_PER_STEP, (i + 1) * ROWS_PER_STEP,
                          lambda j, _: (row_dma(j).wait(), 0)[1], 0)

            e_blk = e_ref[0:1, pl.ds(pl.multiple_of(i * K_BLK, 128), K_BLK)]
            acc_ref[...] += jnp.dot(e_blk, w1_ref[...],
                                    preferred_element_type=jnp.float32)

            @pl.when(i == P1 - 1)
            def _():
                acc_ref[...] = jnp.maximum(acc_ref[...] + b1_ref[...], 0.0)

        @pl.when(i >= P1)
        def _():
            j = i - P1
            slot = lax.rem(j, RING)

            @pl.when(j == 0)
            def _():
                pltpu.make_async_copy(b2_ref, b2v_ref, b2sem).wait()

            @pl.when(j < NV - 1)
            def _():
                w2_dma(j, slot).wait()

            @pl.when(j == NV - 1)
            def _():
                w2_dma_last(slot).wait()

            w2_blk = w2buf[slot]
            b2_blk = b2v_ref[pl.ds(pl.multiple_of(j * V_BLK, 128), V_BLK)]
            z = lax.dot_general(acc_ref[...], w2_blk,
                                (((1,), (1,)), ((), ())),
                                preferred_element_type=jnp.float32) \
                + b2_blk
            logits_ref[pl.ds(j, 1), :] = z
            col = j * V_BLK + lax.broadcasted_iota(jnp.int32, (1, V_BLK), 1)
            zm = jnp.where(col < VOCAB, z, -jnp.inf)
            bm = jnp.max(zm)

            @pl.when(j == 0)
            def _():
                m_ref[0] = bm
                s_ref[0] = jnp.sum(jnp.exp(zm - bm))

            @pl.when(j > 0)
            def _():
                m_old = m_ref[0]
                new_m = jnp.maximum(m_old, bm)
                s_ref[0] = s_ref[0] * jnp.exp(m_old - new_m) + \
                    jnp.sum(jnp.exp(zm - new_m))
                m_ref[0] = new_m

            maybe_issue(j + 1)

            @pl.when(j == NV - 1)
            def _():
                norm = m_ref[0] + jnp.log(s_ref[0])
                for jj in range(NV - 1):
                    out_ref[0:1, jj * V_BLK:(jj + 1) * V_BLK] = \
                        logits_ref[jj:jj + 1, :] - norm
                out_ref[0:1, (NV - 1) * V_BLK:VOCAB] = \
                    logits_ref[NV - 1:NV, 0:LAST_ROWS] - norm

    return pl.pallas_call(
        body,
        grid=(NSTEPS,),
        in_specs=[
            pl.BlockSpec(memory_space=pltpu.SMEM),
            pl.BlockSpec(memory_space=pl.ANY),
            pl.BlockSpec(memory_space=pl.ANY),
            pl.BlockSpec((K_BLK, HIDDEN),
                         lambda i: (jnp.minimum(i, P1 - 1), 0)),
            pl.BlockSpec((HIDDEN,), lambda i: (0,)),
            pl.BlockSpec(memory_space=pl.ANY),
        ],
        out_specs=pl.BlockSpec((1, VOCAB), lambda i: (0, 0)),
        out_shape=jax.ShapeDtypeStruct((1, VOCAB), jnp.float32),
        scratch_shapes=[
            pltpu.VMEM((1, CONTEXT * EMBED), jnp.float32),
            pltpu.VMEM((1, HIDDEN), jnp.float32),
            pltpu.VMEM((NV, V_BLK), jnp.float32),
            pltpu.VMEM((RING, V_BLK, HIDDEN), jnp.float32),
            pltpu.VMEM((VOCAB,), jnp.float32),
            pltpu.SMEM((1,), jnp.float32),
            pltpu.SMEM((1,), jnp.float32),
            pltpu.SMEM((1,), jnp.int32),
            pltpu.SemaphoreType.DMA,
            pltpu.SemaphoreType.DMA((RING,)),
            pltpu.SemaphoreType.DMA,
        ],
    )(idx, table, W2T, W1, b1, b2)


def kernel(idx, table, W1, b1, W2, b2):
    return _fused(idx.astype(jnp.int32), table, W1, b1, W2.T, b2)
